# Initial kernel scaffold; baseline (speedup 1.0000x reference)
import jax
import jax.numpy as jnp
from jax.experimental import pallas as pl


def kernel(x, edge_index, W, b):
    raise NotImplementedError("write your pallas kernel here")


# speedup vs baseline: 3.3436x; 3.3436x over previous
import functools

import jax
import jax.numpy as jnp
from jax import lax
from jax.experimental import pallas as pl
from jax.experimental.pallas import tpu as pltpu
from jax.experimental.pallas import tpu_sc as plsc

N_NODES = 10000
N_EDGES = 320000
D = 128

NC = 2
NS = 16
CHUNK = 128
CHUNKS_PER_TILE = 80
IDX_BLOCK = 16
E_PAD = NC * NS * CHUNKS_PER_TILE * CHUNK
N_PAD = 10240
STRIPE = N_PAD // NS

_sc_mesh = plsc.VectorSubcoreMesh(
    core_axis_name="c", subcore_axis_name="s", num_cores=NC, num_subcores=NS)


@functools.partial(
    pl.kernel,
    out_type=jax.ShapeDtypeStruct((NC, N_PAD, D), jnp.float32),
    mesh=_sc_mesh,
    scratch_types=[
        pltpu.VMEM((IDX_BLOCK, CHUNK), jnp.int32),
        pltpu.VMEM((IDX_BLOCK, CHUNK), jnp.int32),
        pltpu.VMEM((CHUNK, D), jnp.float32),
        pltpu.VMEM((CHUNK, D), jnp.float32),
        pltpu.VMEM_SHARED((N_PAD, D), jnp.float32),
        pltpu.SemaphoreType.DMA,
        pltpu.SemaphoreType.DMA,
    ],
)
def _sc_segment_sum(x_hbm, src_hbm, dst_hbm, out_hbm,
                    src_v, dst_v, rows0, rows1, accum, sg0, sg1):
    c = lax.axis_index("c")
    s = lax.axis_index("s")
    w = c * NS + s

    def _zero_row(i, carry):
        for cc in range(D // 16):
            rows0[i, pl.ds(cc * 16, 16)] = jnp.zeros((16,), jnp.float32)
        return carry
    lax.fori_loop(0, CHUNK, _zero_row, 0)
    for t in range(STRIPE // CHUNK):
        pltpu.sync_copy(rows0, accum.at[pl.ds(s * STRIPE + t * CHUNK, CHUNK)])
    plsc.subcore_barrier()

    for bb in range(CHUNKS_PER_TILE // IDX_BLOCK):
        pltpu.sync_copy(src_hbm.at[w].at[pl.ds(bb * IDX_BLOCK, IDX_BLOCK)], src_v)
        pltpu.sync_copy(dst_hbm.at[w].at[pl.ds(bb * IDX_BLOCK, IDX_BLOCK)], dst_v)

        pltpu.async_copy(x_hbm.at[src_v.at[0]], rows0, sg0)
        pltpu.async_copy(x_hbm.at[src_v.at[1]], rows1, sg1)

        def _step(t, carry):
            j0 = 2 * t
            pltpu.make_async_copy(x_hbm.at[src_v.at[j0]], rows0, sg0).wait()
            pltpu.sync_copy(rows0, accum.at[dst_v.at[j0]], add=True)
            pltpu.async_copy(x_hbm.at[src_v.at[j0 + 2]], rows0, sg0)
            pltpu.make_async_copy(x_hbm.at[src_v.at[j0 + 1]], rows1, sg1).wait()
            pltpu.sync_copy(rows1, accum.at[dst_v.at[j0 + 1]], add=True)
            pltpu.async_copy(x_hbm.at[src_v.at[j0 + 3]], rows1, sg1)
            return carry
        lax.fori_loop(0, IDX_BLOCK // 2 - 1, _step, 0)

        last = IDX_BLOCK - 2
        pltpu.make_async_copy(x_hbm.at[src_v.at[last]], rows0, sg0).wait()
        pltpu.sync_copy(rows0, accum.at[dst_v.at[last]], add=True)
        pltpu.make_async_copy(x_hbm.at[src_v.at[last + 1]], rows1, sg1).wait()
        pltpu.sync_copy(rows1, accum.at[dst_v.at[last + 1]], add=True)

    plsc.subcore_barrier()
    pltpu.sync_copy(accum.at[pl.ds(s * STRIPE, STRIPE)],
                    out_hbm.at[c].at[pl.ds(s * STRIPE, STRIPE)])


def _tc_body(p0_ref, p1_ref, w_ref, b_ref, o_ref):
    acc = p0_ref[0] + p1_ref[0]
    h = lax.dot_general(acc, w_ref[...], (((1,), (1,)), ((), ())),
                        preferred_element_type=jnp.float32)
    o_ref[...] = jnp.maximum(h + b_ref[...], 0.0)


_ROWS_BLK = 1000


def _tc_linear(partials, W, b2d):
    return pl.pallas_call(
        _tc_body,
        grid=(N_NODES // _ROWS_BLK,),
        in_specs=[
            pl.BlockSpec((1, _ROWS_BLK, D), lambda i: (0, i, 0)),
            pl.BlockSpec((1, _ROWS_BLK, D), lambda i: (1, i, 0)),
            pl.BlockSpec((D, D), lambda i: (0, 0)),
            pl.BlockSpec((1, D), lambda i: (0, 0)),
        ],
        out_specs=pl.BlockSpec((_ROWS_BLK, D), lambda i: (i, 0)),
        out_shape=jax.ShapeDtypeStruct((N_NODES, D), jnp.float32),
    )(partials, partials, W, b2d)


def kernel(x, edge_index, W, b):
    src = edge_index[0]
    dst = edge_index[1]
    pad = E_PAD - N_EDGES
    src_p = jnp.concatenate(
        [src, jnp.zeros((pad,), jnp.int32)]).reshape(NC * NS, CHUNKS_PER_TILE, CHUNK)
    dst_p = jnp.concatenate(
        [dst, jnp.full((pad,), N_NODES, jnp.int32)]).reshape(NC * NS, CHUNKS_PER_TILE, CHUNK)
    partials = _sc_segment_sum(x, src_p, dst_p)
    return _tc_linear(partials, W, b.reshape(1, D))


# speedup vs baseline: 3.6116x; 1.0801x over previous
import functools

import jax
import jax.numpy as jnp
from jax import lax
from jax.experimental import pallas as pl
from jax.experimental.pallas import tpu as pltpu
from jax.experimental.pallas import tpu_sc as plsc

N_NODES = 10000
N_EDGES = 320000
D = 128

NC = 2
NS = 16
CHUNK = 128
IDX_BLOCK = 16
C0_BLOCKS = 8
C1_BLOCKS = 2
TOTAL_CHUNKS = NS * (C0_BLOCKS + C1_BLOCKS) * IDX_BLOCK
E_PAD = TOTAL_CHUNKS * CHUNK
N_PAD = 10240
STRIPE = N_PAD // NS

_sc_mesh = plsc.VectorSubcoreMesh(
    core_axis_name="c", subcore_axis_name="s", num_cores=NC, num_subcores=NS)


@functools.partial(
    pl.kernel,
    out_type=jax.ShapeDtypeStruct((NC, N_PAD, D), jnp.float32),
    mesh=_sc_mesh,
    scratch_types=[
        pltpu.VMEM((IDX_BLOCK, CHUNK), jnp.int32),
        pltpu.VMEM((IDX_BLOCK, CHUNK), jnp.int32),
        pltpu.VMEM((CHUNK, D), jnp.float32),
        pltpu.VMEM((CHUNK, D), jnp.float32),
        pltpu.VMEM_SHARED((N_PAD, D), jnp.float32),
        pltpu.SemaphoreType.DMA,
        pltpu.SemaphoreType.DMA,
    ],
)
def _sc_segment_sum(x_hbm, src_hbm, dst_hbm, out_hbm,
                    src_v, dst_v, rows0, rows1, accum, sg0, sg1):
    c = lax.axis_index("c")
    s = lax.axis_index("s")

    def _zero_row(i, carry):
        for cc in range(D // 16):
            rows0[i, pl.ds(cc * 16, 16)] = jnp.zeros((16,), jnp.float32)
        return carry
    lax.fori_loop(0, CHUNK, _zero_row, 0)
    for t in range(STRIPE // CHUNK):
        pltpu.sync_copy(rows0, accum.at[pl.ds(s * STRIPE + t * CHUNK, CHUNK)])
    plsc.subcore_barrier()

    def _run(chunk_base, nblocks):
        for bb in range(nblocks):
            chunk0 = chunk_base + bb * IDX_BLOCK
            pltpu.sync_copy(src_hbm.at[pl.ds(chunk0, IDX_BLOCK)], src_v)
            pltpu.sync_copy(dst_hbm.at[pl.ds(chunk0, IDX_BLOCK)], dst_v)

            pltpu.async_copy(x_hbm.at[src_v.at[0]], rows0, sg0)
            pltpu.async_copy(x_hbm.at[src_v.at[1]], rows1, sg1)

            def _step(t, carry):
                j0 = 2 * t
                pltpu.make_async_copy(x_hbm.at[src_v.at[j0]], rows0, sg0).wait()
                pltpu.sync_copy(rows0, accum.at[dst_v.at[j0]], add=True)
                pltpu.async_copy(x_hbm.at[src_v.at[j0 + 2]], rows0, sg0)
                pltpu.make_async_copy(x_hbm.at[src_v.at[j0 + 1]], rows1, sg1).wait()
                pltpu.sync_copy(rows1, accum.at[dst_v.at[j0 + 1]], add=True)
                pltpu.async_copy(x_hbm.at[src_v.at[j0 + 3]], rows1, sg1)
                return carry
            lax.fori_loop(0, IDX_BLOCK // 2 - 1, _step, 0)

            last = IDX_BLOCK - 2
            pltpu.make_async_copy(x_hbm.at[src_v.at[last]], rows0, sg0).wait()
            pltpu.sync_copy(rows0, accum.at[dst_v.at[last]], add=True)
            pltpu.make_async_copy(x_hbm.at[src_v.at[last + 1]], rows1, sg1).wait()
            pltpu.sync_copy(rows1, accum.at[dst_v.at[last + 1]], add=True)

    @pl.when(c == 0)
    def _():
        _run(s * (C0_BLOCKS * IDX_BLOCK), C0_BLOCKS)

    @pl.when(c == 1)
    def _():
        _run(NS * C0_BLOCKS * IDX_BLOCK + s * (C1_BLOCKS * IDX_BLOCK),
             C1_BLOCKS)

    plsc.subcore_barrier()
    pltpu.sync_copy(accum.at[pl.ds(s * STRIPE, STRIPE)],
                    out_hbm.at[c].at[pl.ds(s * STRIPE, STRIPE)])


def _tc_body(p0_ref, p1_ref, w_ref, b_ref, o_ref):
    acc = p0_ref[0] + p1_ref[0]
    h = lax.dot_general(acc, w_ref[...], (((1,), (1,)), ((), ())),
                        preferred_element_type=jnp.float32)
    o_ref[...] = jnp.maximum(h + b_ref[...], 0.0)


_ROWS_BLK = 1000


def _tc_linear(partials, W, b2d):
    return pl.pallas_call(
        _tc_body,
        grid=(N_NODES // _ROWS_BLK,),
        in_specs=[
            pl.BlockSpec((1, _ROWS_BLK, D), lambda i: (0, i, 0)),
            pl.BlockSpec((1, _ROWS_BLK, D), lambda i: (1, i, 0)),
            pl.BlockSpec((D, D), lambda i: (0, 0)),
            pl.BlockSpec((1, D), lambda i: (0, 0)),
        ],
        out_specs=pl.BlockSpec((_ROWS_BLK, D), lambda i: (i, 0)),
        out_shape=jax.ShapeDtypeStruct((N_NODES, D), jnp.float32),
    )(partials, partials, W, b2d)


def kernel(x, edge_index, W, b):
    src = edge_index[0]
    dst = edge_index[1]
    pad = E_PAD - N_EDGES
    src_p = jnp.concatenate(
        [src, jnp.zeros((pad,), jnp.int32)]).reshape(TOTAL_CHUNKS, CHUNK)
    dst_p = jnp.concatenate(
        [dst, jnp.full((pad,), N_NODES, jnp.int32)]).reshape(TOTAL_CHUNKS, CHUNK)
    partials = _sc_segment_sum(x, src_p, dst_p)
    return _tc_linear(partials, W, b.reshape(1, D))
